# baseline (speedup 1.0000x reference)
import jax
import jax.numpy as jnp
from jax.experimental import pallas as pl
from jax.experimental.pallas import tpu as pltpu


def _bigru_kernel(x_ref, wih_ref, whh_ref, bih_ref, bhh_ref,
                  out_ref, hid_ref, xp_ref):
    B2, T, D = x_ref.shape
    Hp = whh_ref.shape[0] // 2
    G = 3 * Hp
    H = hid_ref.shape[2]

    x2 = x_ref[...].reshape(B2 * T, D)
    xp = jnp.dot(x2, wih_ref[...], preferred_element_type=jnp.float32) + bih_ref[...]
    xp_ref[...] = xp.reshape(B2, T, 2 * G)

    whh = whh_ref[...]
    whh_f = whh[0:Hp, 0:G]
    whh_b = whh[Hp:2 * Hp, G:2 * G]
    bhh = bhh_ref[...]
    bhh_f = bhh[:, 0:G]
    bhh_b = bhh[:, G:2 * G]

    def gates(xg, hg, h_prev):
        r = jax.nn.sigmoid(xg[:, 0:Hp] + hg[:, 0:Hp])
        z = jax.nn.sigmoid(xg[:, Hp:2 * Hp] + hg[:, Hp:2 * Hp])
        n = jnp.tanh(xg[:, 2 * Hp:3 * Hp] + r * hg[:, 2 * Hp:3 * Hp])
        return (1.0 - z) * n + z * h_prev

    h_f = jnp.zeros((B2, Hp), jnp.float32)
    h_b = jnp.zeros((B2, Hp), jnp.float32)

    for t in range(T):
        hp_f = jnp.dot(h_f, whh_f, preferred_element_type=jnp.float32) + bhh_f
        hp_b = jnp.dot(h_b, whh_b, preferred_element_type=jnp.float32) + bhh_b

        xt_f = xp_ref[:, pl.ds(t, 1), 0:G][:, 0, :]
        xt_b = xp_ref[:, pl.ds(T - 1 - t, 1), G:2 * G][:, 0, :]

        h_f = gates(xt_f, hp_f, h_f)
        h_b = gates(xt_b, hp_b, h_b)

        out_ref[:, pl.ds(t, 1), 0:H] = h_f[:, None, 0:H]
        out_ref[:, pl.ds(T - 1 - t, 1), H:2 * H] = h_b[:, None, 0:H]

    hid_ref[0, :, :] = h_f[:, 0:H]
    hid_ref[1, :, :] = h_b[:, 0:H]


def _bigru(embedded, wih_c, whh_c, bih_c, bhh_c, hidden_per_dir):
    B, T, D = embedded.shape
    H = int(hidden_per_dir)
    Hp = whh_c.shape[0] // 2
    G = 3 * Hp
    NB = 2
    B2 = B // NB

    cost = pl.CostEstimate(
        flops=2 * B * T * D * (2 * G) + 2 * T * B * 2 * (Hp * G),
        transcendentals=3 * T * B * 2 * Hp,
        bytes_accessed=4 * (embedded.size + wih_c.size + whh_c.size
                            + bih_c.size + bhh_c.size + B * T * 2 * H + 2 * B * H),
    )

    return pl.pallas_call(
        _bigru_kernel,
        out_shape=(
            jax.ShapeDtypeStruct((B, T, 2 * H), jnp.float32),
            jax.ShapeDtypeStruct((2, B, H), jnp.float32),
        ),
        grid=(NB,),
        in_specs=[
            pl.BlockSpec((B2, T, D), lambda i: (i, 0, 0)),
            pl.BlockSpec((D, 2 * G), lambda i: (0, 0)),
            pl.BlockSpec((2 * Hp, 2 * G), lambda i: (0, 0)),
            pl.BlockSpec((1, 2 * G), lambda i: (0, 0)),
            pl.BlockSpec((1, 2 * G), lambda i: (0, 0)),
        ],
        out_specs=(
            pl.BlockSpec((B2, T, 2 * H), lambda i: (i, 0, 0)),
            pl.BlockSpec((2, B2, H), lambda i: (0, i, 0)),
        ),
        scratch_shapes=[pltpu.VMEM((B2, T, 2 * G), jnp.float32)],
        compiler_params=pltpu.CompilerParams(
            dimension_semantics=("parallel",)),
        cost_estimate=cost,
    )(embedded, wih_c, whh_c, bih_c, bhh_c)


def kernel(input_seq, embedding, wih_t, whh_t, bih, bhh,
           wih_c, whh_c, bih_c, bhh_c):
    embedded = embedding[input_seq]
    H = whh_t.shape[1]
    return _bigru(embedded, wih_c, whh_c, bih_c, bhh_c, H)


# speedup vs baseline: 1.2389x; 1.2389x over previous
import jax
import jax.numpy as jnp
from jax.experimental import pallas as pl
from jax.experimental.pallas import tpu as pltpu


def _bigru_kernel(x_ref, wih_ref, whh_ref, bih_ref, bhh_ref,
                  out_ref, hid_ref, xp_ref):
    T, B2, D = x_ref.shape
    Hp = whh_ref.shape[0] // 2
    G = 3 * Hp
    H = hid_ref.shape[2]

    x2 = x_ref[...].reshape(T * B2, D)
    xp = jnp.dot(x2, wih_ref[...], preferred_element_type=jnp.float32) + bih_ref[...]
    xp_ref[...] = xp.reshape(T, B2, 2 * G)

    whh = whh_ref[...]
    whh_f = whh[0:Hp, 0:G]
    whh_b = whh[Hp:2 * Hp, G:2 * G]
    bhh = bhh_ref[...]
    bhh_f = bhh[:, 0:G]
    bhh_b = bhh[:, G:2 * G]

    def gates(xg, hg, h_prev):
        r = jax.nn.sigmoid(xg[:, 0:Hp] + hg[:, 0:Hp])
        z = jax.nn.sigmoid(xg[:, Hp:2 * Hp] + hg[:, Hp:2 * Hp])
        n = jnp.tanh(xg[:, 2 * Hp:3 * Hp] + r * hg[:, 2 * Hp:3 * Hp])
        return (1.0 - z) * n + z * h_prev

    h_f = jnp.zeros((B2, Hp), jnp.float32)
    h_b = jnp.zeros((B2, Hp), jnp.float32)

    for t in range(T):
        hp_f = jnp.dot(h_f, whh_f, preferred_element_type=jnp.float32) + bhh_f
        hp_b = jnp.dot(h_b, whh_b, preferred_element_type=jnp.float32) + bhh_b

        xt_f = xp_ref[t, :, 0:G]
        xt_b = xp_ref[T - 1 - t, :, G:2 * G]

        h_f = gates(xt_f, hp_f, h_f)
        h_b = gates(xt_b, hp_b, h_b)

        out_ref[t, :, 0:H] = h_f[:, 0:H]
        out_ref[T - 1 - t, :, H:2 * H] = h_b[:, 0:H]

    hid_ref[0, :, :] = h_f[:, 0:H]
    hid_ref[1, :, :] = h_b[:, 0:H]


def _bigru(embedded_tm, wih_c, whh_c, bih_c, bhh_c, hidden_per_dir):
    T, B, D = embedded_tm.shape
    H = int(hidden_per_dir)
    Hp = whh_c.shape[0] // 2
    G = 3 * Hp
    NB = 2
    B2 = B // NB

    cost = pl.CostEstimate(
        flops=2 * B * T * D * (2 * G) + 2 * T * B * 2 * (Hp * G),
        transcendentals=3 * T * B * 2 * Hp,
        bytes_accessed=4 * (embedded_tm.size + wih_c.size + whh_c.size
                            + bih_c.size + bhh_c.size + B * T * 2 * H + 2 * B * H),
    )

    return pl.pallas_call(
        _bigru_kernel,
        out_shape=(
            jax.ShapeDtypeStruct((T, B, 2 * H), jnp.float32),
            jax.ShapeDtypeStruct((2, B, H), jnp.float32),
        ),
        grid=(NB,),
        in_specs=[
            pl.BlockSpec((T, B2, D), lambda i: (0, i, 0)),
            pl.BlockSpec((D, 2 * G), lambda i: (0, 0)),
            pl.BlockSpec((2 * Hp, 2 * G), lambda i: (0, 0)),
            pl.BlockSpec((1, 2 * G), lambda i: (0, 0)),
            pl.BlockSpec((1, 2 * G), lambda i: (0, 0)),
        ],
        out_specs=(
            pl.BlockSpec((T, B2, 2 * H), lambda i: (0, i, 0)),
            pl.BlockSpec((2, B2, H), lambda i: (0, i, 0)),
        ),
        scratch_shapes=[pltpu.VMEM((T, B2, 2 * G), jnp.float32)],
        compiler_params=pltpu.CompilerParams(
            dimension_semantics=("parallel",)),
        cost_estimate=cost,
    )(embedded_tm, wih_c, whh_c, bih_c, bhh_c)


def kernel(input_seq, embedding, wih_t, whh_t, bih, bhh,
           wih_c, whh_c, bih_c, bhh_c):
    embedded_tm = embedding[input_seq.T]
    H = whh_t.shape[1]
    out_tm, hidden = _bigru(embedded_tm, wih_c, whh_c, bih_c, bhh_c, H)
    return jnp.transpose(out_tm, (1, 0, 2)), hidden


# speedup vs baseline: 1.3876x; 1.1200x over previous
import jax
import jax.numpy as jnp
from jax.experimental import pallas as pl
from jax.experimental.pallas import tpu as pltpu


def _bigru_kernel(x_ref, wih_ref, whh_ref, bih_ref, bhh_ref,
                  out_ref, hid_ref, xp_ref):
    T, B2, D = x_ref.shape
    Hp = whh_ref.shape[0] // 2
    G = 3 * Hp
    H = hid_ref.shape[2]

    x2 = x_ref[...].reshape(T * B2, D)
    xp = jnp.dot(x2, wih_ref[...], preferred_element_type=jnp.float32) + bih_ref[...]
    xp_ref[...] = xp.reshape(T, B2, 2 * G)

    whh = whh_ref[...]
    whh_f = whh[0:Hp, 0:G]
    whh_b = whh[Hp:2 * Hp, G:2 * G]
    bhh = bhh_ref[...]
    bhh_f = bhh[:, 0:G]
    bhh_b = bhh[:, G:2 * G]

    def gates(xg, hg, h_prev):
        r = jax.nn.sigmoid(xg[:, 0:Hp] + hg[:, 0:Hp])
        z = jax.nn.sigmoid(xg[:, Hp:2 * Hp] + hg[:, Hp:2 * Hp])
        n = jnp.tanh(xg[:, 2 * Hp:3 * Hp] + r * hg[:, 2 * Hp:3 * Hp])
        return (1.0 - z) * n + z * h_prev

    h_f = jnp.zeros((B2, Hp), jnp.float32)
    h_b = jnp.zeros((B2, Hp), jnp.float32)

    for t in range(T):
        hp_f = jnp.dot(h_f, whh_f, preferred_element_type=jnp.float32) + bhh_f
        hp_b = jnp.dot(h_b, whh_b, preferred_element_type=jnp.float32) + bhh_b

        xt_f = xp_ref[t, :, 0:G]
        xt_b = xp_ref[T - 1 - t, :, G:2 * G]

        h_f = gates(xt_f, hp_f, h_f)
        h_b = gates(xt_b, hp_b, h_b)

        out_ref[:, pl.ds(t, 1), 0:H] = h_f[:, None, 0:H]
        out_ref[:, pl.ds(T - 1 - t, 1), H:2 * H] = h_b[:, None, 0:H]

    hid_ref[0, :, :] = h_f[:, 0:H]
    hid_ref[1, :, :] = h_b[:, 0:H]


def _bigru(embedded_tm, wih_c, whh_c, bih_c, bhh_c, hidden_per_dir):
    T, B, D = embedded_tm.shape
    H = int(hidden_per_dir)
    Hp = whh_c.shape[0] // 2
    G = 3 * Hp
    NB = 2
    B2 = B // NB

    cost = pl.CostEstimate(
        flops=2 * B * T * D * (2 * G) + 2 * T * B * 2 * (Hp * G),
        transcendentals=3 * T * B * 2 * Hp,
        bytes_accessed=4 * (embedded_tm.size + wih_c.size + whh_c.size
                            + bih_c.size + bhh_c.size + B * T * 2 * H + 2 * B * H),
    )

    return pl.pallas_call(
        _bigru_kernel,
        out_shape=(
            jax.ShapeDtypeStruct((B, T, 2 * H), jnp.float32),
            jax.ShapeDtypeStruct((2, B, H), jnp.float32),
        ),
        grid=(NB,),
        in_specs=[
            pl.BlockSpec((T, B2, D), lambda i: (0, i, 0)),
            pl.BlockSpec((D, 2 * G), lambda i: (0, 0)),
            pl.BlockSpec((2 * Hp, 2 * G), lambda i: (0, 0)),
            pl.BlockSpec((1, 2 * G), lambda i: (0, 0)),
            pl.BlockSpec((1, 2 * G), lambda i: (0, 0)),
        ],
        out_specs=(
            pl.BlockSpec((B2, T, 2 * H), lambda i: (i, 0, 0)),
            pl.BlockSpec((2, B2, H), lambda i: (0, i, 0)),
        ),
        scratch_shapes=[pltpu.VMEM((T, B2, 2 * G), jnp.float32)],
        compiler_params=pltpu.CompilerParams(
            dimension_semantics=("parallel",)),
        cost_estimate=cost,
    )(embedded_tm, wih_c, whh_c, bih_c, bhh_c)


def kernel(input_seq, embedding, wih_t, whh_t, bih, bhh,
           wih_c, whh_c, bih_c, bhh_c):
    embedded_tm = embedding[input_seq.T]
    H = whh_t.shape[1]
    return _bigru(embedded_tm, wih_c, whh_c, bih_c, bhh_c, H)
